# baseline (speedup 1.0000x reference)
import functools

import jax
import jax.numpy as jnp
from jax import lax
from jax.experimental import pallas as pl
from jax.experimental.pallas import tpu as pltpu
from jax.experimental.pallas import tpu_sc as plsc

LANES = 16


@functools.lru_cache(maxsize=None)
def _build_sc_call(batch: int, fields: int, emb_dim: int):
    NW = 32
    bw = batch // NW
    chunk = 128
    tb_w = bw // chunk
    n_units = fields * tb_w
    te_n = emb_dim // 8
    assert bw * NW == batch and tb_w * chunk == bw and n_units % 2 == 0

    mesh = plsc.VectorSubcoreMesh(core_axis_name="c", subcore_axis_name="s")

    @functools.partial(
        pl.kernel,
        out_type=jax.ShapeDtypeStruct(
            (fields, te_n, batch // chunk, 8, chunk), jnp.float32),
        mesh=mesh,
        compiler_params=pltpu.CompilerParams(use_tc_tiling_on_sc=False,
                                             needs_layout_passes=False),
        scratch_types=[
            pltpu.VMEM((fields, bw), jnp.int32),
            pltpu.VMEM((fields, bw), jnp.int32),
            pltpu.VMEM((chunk, emb_dim), jnp.float32),
            pltpu.VMEM((chunk, emb_dim), jnp.float32),
            pltpu.VMEM((chunk, emb_dim), jnp.float32),
            pltpu.VMEM((chunk, emb_dim), jnp.float32),
            pltpu.VMEM((te_n, 1, 8, chunk), jnp.float32),
            pltpu.VMEM((te_n, 1, 8, chunk), jnp.float32),
            pltpu.SemaphoreType.DMA,
            pltpu.SemaphoreType.DMA,
            pltpu.SemaphoreType.DMA,
            pltpu.SemaphoreType.DMA,
        ],
    )
    def sc_call(idx1_hbm, idx2_hbm, emb1_hbm, emb2_hbm, out_hbm,
                idx1_v, idx2_v, r1a, r1b, r2a, r2b, ta, tb,
                sg_a, sg_b, so_a, so_b):
        r1 = (r1a, r1b)
        r2 = (r2a, r2b)
        tbuf = (ta, tb)
        sg = (sg_a, sg_b)
        so = (so_a, so_b)

        wid = lax.axis_index("s") * 2 + lax.axis_index("c")
        b0 = wid * bw
        pltpu.sync_copy(idx1_hbm.at[pl.ds(0, fields), pl.ds(b0, bw)], idx1_v)
        pltpu.sync_copy(idx2_hbm.at[pl.ds(0, fields), pl.ds(b0, bw)], idx2_v)

        iota16 = lax.iota(jnp.int32, 16)
        lconst = [iota16 + lg * 16 for lg in range(chunk // 16)]

        def fire_gathers(u, slot):
            f = u // tb_w
            off = (u % tb_w) * chunk
            pltpu.async_copy(
                emb1_hbm.at[idx1_v.at[f, pl.ds(off, chunk)]],
                r1[slot], sg[slot])
            pltpu.async_copy(
                emb2_hbm.at[idx2_v.at[f, pl.ds(off, chunk)]],
                r2[slot], sg[slot])

        def drain_gathers(slot):
            pltpu.make_async_copy(
                emb1_hbm.at[pl.ds(0, chunk)], r1[slot], sg[slot]).wait()
            pltpu.make_async_copy(
                emb1_hbm.at[pl.ds(0, chunk)], r2[slot], sg[slot]).wait()

        def drain_out(slot):
            pltpu.make_async_copy(
                tbuf[slot], out_hbm.at[0, pl.ds(0, te_n), pl.ds(0, 1)],
                so[slot]).wait()

        def compute(slot):
            a, b2, t = r1[slot], r2[slot], tbuf[slot]
            for e in range(emb_dim):
                te, s = divmod(e, 8)
                ev = jnp.full((16,), e, jnp.int32)
                for lg in range(chunk // 16):
                    v1 = plsc.load_gather(a, [lconst[lg], ev])
                    v2 = plsc.load_gather(b2, [lconst[lg], ev])
                    t[te, 0, s, pl.ds(lg * 16, 16)] = v1 * v2

        fire_gathers(0, 0)
        fire_gathers(1, 1)

        def step(i, carry):
            for slot in range(2):
                u = 2 * i + slot
                f = u // tb_w
                tbi = wid * tb_w + u % tb_w
                drain_gathers(slot)

                @pl.when(i > 0)
                def _():
                    drain_out(slot)

                compute(slot)

                @pl.when(u + 2 < n_units)
                def _():
                    fire_gathers(u + 2, slot)

                pltpu.async_copy(
                    tbuf[slot],
                    out_hbm.at[f, pl.ds(0, te_n), pl.ds(tbi, 1)],
                    so[slot])
            return carry

        lax.fori_loop(0, n_units // 2, step, 0)
        drain_out(0)
        drain_out(1)

    return sc_call


TR_C = 4096


@functools.lru_cache(maxsize=None)
def _build_transpose_call(n_tab: int, emb_dim: int):
    rows_per_srow = 128 // emb_dim
    C = TR_C
    grid = (n_tab + 4 * C - 1) // (4 * C)
    s_rows = grid * C
    max_blk = (n_tab + C - 1) // C - 1

    def body(*refs):
        in_refs, out_ref = refs[:rows_per_srow], refs[rows_per_srow]
        for j in range(rows_per_srow):
            out_ref[:, j * emb_dim:(j + 1) * emb_dim] = jnp.transpose(
                in_refs[j][...])

    return pl.pallas_call(
        body,
        grid=(grid,),
        in_specs=[
            pl.BlockSpec((emb_dim, C),
                         lambda i, j=j: (0, jnp.minimum(4 * i + j, max_blk)))
            for j in range(rows_per_srow)
        ],
        out_specs=pl.BlockSpec((C, 128), lambda i: (i, 0)),
        out_shape=jax.ShapeDtypeStruct((s_rows, 128), jnp.float32),
    )


def kernel(first, second, emb1, emb2):
    b, f = first.shape
    emb_dim = emb1.shape[1]
    n_tab = emb1.shape[0]
    C = TR_C

    def remap(i):
        i = i.astype(jnp.int32)
        return ((i & ~(4 * C - 1)) + 4 * (i & (C - 1))
                + ((i >> C.bit_length() - 1) & 3))

    idx1 = remap(first).T
    idx2 = remap(second).T
    tr_call = _build_transpose_call(n_tab, emb_dim)
    e1t, e2t = emb1.T, emb2.T
    s1 = tr_call(e1t, e1t, e1t, e1t)
    s2 = tr_call(e2t, e2t, e2t, e2t)
    e1 = s1.reshape(-1, emb_dim)
    e2 = s2.reshape(-1, emb_dim)
    sc_call = _build_sc_call(b, f, emb_dim)
    out5 = sc_call(idx1, idx2, e1, e2)
    return jnp.transpose(out5, (2, 4, 0, 1, 3)).reshape(b, f, emb_dim)


# speedup vs baseline: 1.3914x; 1.3914x over previous
import functools

import jax
import jax.numpy as jnp
from jax import lax
from jax.experimental import pallas as pl
from jax.experimental.pallas import tpu as pltpu
from jax.experimental.pallas import tpu_sc as plsc

LANES = 16


@functools.lru_cache(maxsize=None)
def _build_sc_call(batch: int, fields: int, emb_dim: int):
    NW = 32
    bw = batch // NW
    chunk = 128
    tb_w = bw // chunk
    n_units = fields * tb_w
    te_n = emb_dim // 8
    assert bw * NW == batch and tb_w * chunk == bw and n_units % 2 == 0

    mesh = plsc.VectorSubcoreMesh(core_axis_name="c", subcore_axis_name="s")

    @functools.partial(
        pl.kernel,
        out_type=jax.ShapeDtypeStruct(
            (fields, te_n, batch // chunk, 8, chunk), jnp.float32),
        mesh=mesh,
        compiler_params=pltpu.CompilerParams(use_tc_tiling_on_sc=False,
                                             needs_layout_passes=False),
        scratch_types=[
            pltpu.VMEM((fields, bw), jnp.int32),
            pltpu.VMEM((fields, bw), jnp.int32),
            pltpu.VMEM((chunk, emb_dim), jnp.float32),
            pltpu.VMEM((chunk, emb_dim), jnp.float32),
            pltpu.VMEM((chunk, emb_dim), jnp.float32),
            pltpu.VMEM((chunk, emb_dim), jnp.float32),
            pltpu.VMEM((te_n, 8, chunk), jnp.float32),
            pltpu.VMEM((te_n, 8, chunk), jnp.float32),
            pltpu.SemaphoreType.DMA,
            pltpu.SemaphoreType.DMA,
            pltpu.SemaphoreType.DMA,
            pltpu.SemaphoreType.DMA,
        ],
    )
    def sc_call(idx1_hbm, idx2_hbm, emb1_hbm, emb2_hbm, out_hbm,
                idx1_v, idx2_v, r1a, r1b, r2a, r2b, ta, tb,
                sg_a, sg_b, so_a, so_b):
        r1 = (r1a, r1b)
        r2 = (r2a, r2b)
        tbuf = (ta, tb)
        sg = (sg_a, sg_b)
        so = (so_a, so_b)

        wid = lax.axis_index("s") * 2 + lax.axis_index("c")
        b0 = wid * bw
        pltpu.sync_copy(idx1_hbm.at[pl.ds(0, fields), pl.ds(b0, bw)], idx1_v)
        pltpu.sync_copy(idx2_hbm.at[pl.ds(0, fields), pl.ds(b0, bw)], idx2_v)

        iota16 = lax.iota(jnp.int32, 16)
        sev = iota16 & 7
        tev = [(iota16 + 16 * h) >> 3 for h in range(emb_dim // 16)]

        def fire_gathers(u, slot):
            f = u // tb_w
            off = (u % tb_w) * chunk
            pltpu.async_copy(
                emb1_hbm.at[idx1_v.at[f, pl.ds(off, chunk)]],
                r1[slot], sg[slot])
            pltpu.async_copy(
                emb2_hbm.at[idx2_v.at[f, pl.ds(off, chunk)]],
                r2[slot], sg[slot])

        def drain_gathers(slot):
            pltpu.make_async_copy(
                emb1_hbm.at[pl.ds(0, chunk)], r1[slot], sg[slot]).wait()
            pltpu.make_async_copy(
                emb1_hbm.at[pl.ds(0, chunk)], r2[slot], sg[slot]).wait()

        def drain_out(slot):
            pltpu.make_async_copy(
                tbuf[slot], out_hbm.at[0, 0, pl.ds(0, te_n)],
                so[slot]).wait()

        def compute(slot):
            a, b2, t = r1[slot], r2[slot], tbuf[slot]

            def row(l, c):
                lv = jnp.full((16,), l, jnp.int32)
                for h in range(emb_dim // 16):
                    v1 = a[l, pl.ds(h * 16, 16)]
                    v2 = b2[l, pl.ds(h * 16, 16)]
                    plsc.store_scatter(t, [tev[h], sev, lv], v1 * v2)
                return c

            lax.fori_loop(0, chunk, row, 0, unroll=4)

        fire_gathers(0, 0)
        fire_gathers(1, 1)

        def step(i, carry):
            for slot in range(2):
                u = 2 * i + slot
                f = u // tb_w
                tbi = wid * tb_w + u % tb_w
                drain_gathers(slot)

                @pl.when(i > 0)
                def _():
                    drain_out(slot)

                compute(slot)

                @pl.when(u + 2 < n_units)
                def _():
                    fire_gathers(u + 2, slot)

                for te in range(te_n):
                    pltpu.async_copy(
                        tbuf[slot].at[pl.ds(te, 1)],
                        out_hbm.at[f, te, pl.ds(tbi, 1)],
                        so[slot])
            return carry

        lax.fori_loop(0, n_units // 2, step, 0)
        drain_out(0)
        drain_out(1)

    return sc_call


TR_C = 4096


@functools.lru_cache(maxsize=None)
def _build_transpose_call(n_tab: int, emb_dim: int):
    rows_per_srow = 128 // emb_dim
    C = TR_C
    grid = (n_tab + 4 * C - 1) // (4 * C)
    s_rows = grid * C
    max_blk = (n_tab + C - 1) // C - 1

    def body(*refs):
        in_refs, out_ref = refs[:rows_per_srow], refs[rows_per_srow]
        for j in range(rows_per_srow):
            out_ref[:, j * emb_dim:(j + 1) * emb_dim] = jnp.transpose(
                in_refs[j][...])

    return pl.pallas_call(
        body,
        grid=(grid,),
        in_specs=[
            pl.BlockSpec((emb_dim, C),
                         lambda i, j=j: (0, jnp.minimum(4 * i + j, max_blk)))
            for j in range(rows_per_srow)
        ],
        out_specs=pl.BlockSpec((C, 128), lambda i: (i, 0)),
        out_shape=jax.ShapeDtypeStruct((s_rows, 128), jnp.float32),
    )


def kernel(first, second, emb1, emb2):
    b, f = first.shape
    emb_dim = emb1.shape[1]
    n_tab = emb1.shape[0]
    C = TR_C

    def remap(i):
        i = i.astype(jnp.int32)
        return ((i & ~(4 * C - 1)) + 4 * (i & (C - 1))
                + ((i >> C.bit_length() - 1) & 3))

    idx1 = remap(first).T
    idx2 = remap(second).T
    tr_call = _build_transpose_call(n_tab, emb_dim)
    e1t, e2t = emb1.T, emb2.T
    s1 = tr_call(e1t, e1t, e1t, e1t)
    s2 = tr_call(e2t, e2t, e2t, e2t)
    e1 = s1.reshape(-1, emb_dim)
    e2 = s2.reshape(-1, emb_dim)
    sc_call = _build_sc_call(b, f, emb_dim)
    out5 = sc_call(idx1, idx2, e1, e2)
    return jnp.transpose(out5, (2, 4, 0, 1, 3)).reshape(b, f, emb_dim)
